# baseline (speedup 1.0000x reference)
import functools

import jax
import jax.numpy as jnp
from jax.experimental import pallas as pl
from jax.experimental.pallas import tpu as pltpu

RADIUS = 2.0
Y_ACC = 0.001
X_ACC = 0.001
NUM_DIMS = 8
TEMP = 50.0
EPS = 0.0001
BATCH = 2048
NY = int(2 * RADIUS / Y_ACC) + 1
NX = int(2 * RADIUS / X_ACC) + 1
NY_PAD = 4096
L2E = 1.4426950408889634


def _fsm_kernel(x_ref, b_ref, out_ref):
    bt = b_ref[...]

    tl2e = TEMP * L2E
    tm = TEMP * (1.0 - jnp.min(bt, axis=-1, keepdims=True))
    c2 = L2E * (TEMP * bt + tm)

    y = (
        jax.lax.broadcasted_iota(jnp.int32, (1, NY_PAD), 1).astype(jnp.float32)
        * Y_ACC
        - RADIUS
    )

    def body(i, _):
        x = x_ref[i]
        xp = jnp.clip(x, -RADIUS + EPS, RADIUS - EPS)
        idx = jnp.round((xp + RADIUS) / (2.0 * RADIUS) * (NX - 1))
        xg = -RADIUS + idx * (2.0 * RADIUS / (NX - 1))

        d = xg - y
        q = jnp.exp2(d * d * (-L2E))
        s = q - bt
        e = jnp.exp2(q * tl2e - c2)
        num = jnp.sum(e * s, axis=-1, keepdims=True)
        den = jnp.sum(e, axis=-1, keepdims=True)
        out_ref[pl.ds(i, 1), :] = jnp.sum(num / den, axis=0, keepdims=True)
        return 0

    jax.lax.fori_loop(0, BATCH, body, 0)


@jax.jit
def kernel(X, theta):
    b = jnp.concatenate(
        [jnp.zeros((1, NUM_DIMS), jnp.float32), theta], axis=0
    )
    bt = jnp.full((NUM_DIMS, NY_PAD), 1e30, jnp.float32).at[:, :NY].set(b.T)

    out = pl.pallas_call(
        _fsm_kernel,
        grid=(1,),
        in_specs=[
            pl.BlockSpec((BATCH, NUM_DIMS, 1), lambda i: (0, 0, 0)),
            pl.BlockSpec((NUM_DIMS, NY_PAD), lambda i: (0, 0)),
        ],
        out_specs=pl.BlockSpec((BATCH, 1), lambda i: (0, 0)),
        out_shape=jax.ShapeDtypeStruct((BATCH, 1), jnp.float32),
        compiler_params=pltpu.CompilerParams(
            dimension_semantics=("arbitrary",),
        ),
    )(X.reshape(BATCH, NUM_DIMS, 1), bt)
    return out.reshape(BATCH)


# speedup vs baseline: 3.8900x; 3.8900x over previous
import functools

import jax
import jax.numpy as jnp
from jax.experimental import pallas as pl
from jax.experimental.pallas import tpu as pltpu

RADIUS = 2.0
Y_ACC = 0.001
X_ACC = 0.001
NUM_DIMS = 8
TEMP = 50.0
EPS = 0.0001
BATCH = 2048
NY = int(2 * RADIUS / Y_ACC) + 1
NX = int(2 * RADIUS / X_ACC) + 1
NY_PAD = 4096
L2E = 1.4426950408889634
BLK_B = 64


def _fsm_kernel(x_ref, b_ref, y_ref, out_ref):
    x = x_ref[...]
    xp = jnp.clip(x, -RADIUS + EPS, RADIUS - EPS)
    idx = jnp.round((xp + RADIUS) / (2.0 * RADIUS) * (NX - 1))
    xg = -RADIUS + idx * (2.0 * RADIUS / (NX - 1))

    y = y_ref[...]
    bt = b_ref[...]

    tl2e = TEMP * L2E
    ltm = tl2e * (1.0 - jnp.min(bt, axis=-1, keepdims=True))

    d = xg[:, :, None] - y[None, :, :]
    s = jnp.exp2(d * d * (-L2E)) - bt[None, :, :]
    e = jnp.exp2(s * tl2e - ltm[None, :, :])
    num = jnp.sum(e * s, axis=-1)
    den = jnp.sum(e, axis=-1)
    out_ref[...] = jnp.sum(num / den, axis=-1, keepdims=True)


@jax.jit
def kernel(X, theta):
    y_grid = jnp.linspace(-RADIUS, RADIUS, NY, dtype=jnp.float32)
    y_pad = jnp.zeros((1, NY_PAD), jnp.float32).at[0, :NY].set(y_grid)
    b = jnp.concatenate(
        [jnp.zeros((1, NUM_DIMS), jnp.float32), theta], axis=0
    )
    bt = jnp.full((NUM_DIMS, NY_PAD), 1e30, jnp.float32).at[:, :NY].set(b.T)

    grid = BATCH // BLK_B
    out = pl.pallas_call(
        _fsm_kernel,
        grid=(grid,),
        in_specs=[
            pl.BlockSpec((BLK_B, NUM_DIMS), lambda i: (i, 0)),
            pl.BlockSpec((NUM_DIMS, NY_PAD), lambda i: (0, 0)),
            pl.BlockSpec((1, NY_PAD), lambda i: (0, 0)),
        ],
        out_specs=pl.BlockSpec((BLK_B, 1), lambda i: (i, 0)),
        out_shape=jax.ShapeDtypeStruct((BATCH, 1), jnp.float32),
        compiler_params=pltpu.CompilerParams(
            dimension_semantics=("arbitrary",),
        ),
    )(X, bt, y_pad)
    return out.reshape(BATCH)


# speedup vs baseline: 3.9008x; 1.0028x over previous
import functools

import jax
import jax.numpy as jnp
from jax.experimental import pallas as pl
from jax.experimental.pallas import tpu as pltpu

RADIUS = 2.0
Y_ACC = 0.001
X_ACC = 0.001
NUM_DIMS = 8
TEMP = 50.0
EPS = 0.0001
BATCH = 2048
NY = int(2 * RADIUS / Y_ACC) + 1
NX = int(2 * RADIUS / X_ACC) + 1
NY_PAD = 4096
L2E = 1.4426950408889634
BLK_B = 128


def _fsm_kernel(x_ref, b_ref, y_ref, out_ref):
    x = x_ref[...]
    xp = jnp.clip(x, -RADIUS + EPS, RADIUS - EPS)
    idx = jnp.round((xp + RADIUS) / (2.0 * RADIUS) * (NX - 1))
    xg = -RADIUS + idx * (2.0 * RADIUS / (NX - 1))

    y = y_ref[...]
    bt = b_ref[...]

    tl2e = TEMP * L2E
    ltm = tl2e * (1.0 - jnp.min(bt, axis=-1, keepdims=True))

    d = xg[:, :, None] - y[None, :, :]
    s = jnp.exp2(d * d * (-L2E)) - bt[None, :, :]
    e = jnp.exp2(s * tl2e - ltm[None, :, :])
    num = jnp.sum(e * s, axis=-1)
    den = jnp.sum(e, axis=-1)
    out_ref[...] = jnp.sum(num / den, axis=-1, keepdims=True)


@jax.jit
def kernel(X, theta):
    y_grid = jnp.linspace(-RADIUS, RADIUS, NY, dtype=jnp.float32)
    y_pad = jnp.zeros((1, NY_PAD), jnp.float32).at[0, :NY].set(y_grid)
    b = jnp.concatenate(
        [jnp.zeros((1, NUM_DIMS), jnp.float32), theta], axis=0
    )
    bt = jnp.full((NUM_DIMS, NY_PAD), 1e30, jnp.float32).at[:, :NY].set(b.T)

    grid = BATCH // BLK_B
    out = pl.pallas_call(
        _fsm_kernel,
        grid=(grid,),
        in_specs=[
            pl.BlockSpec((BLK_B, NUM_DIMS), lambda i: (i, 0)),
            pl.BlockSpec((NUM_DIMS, NY_PAD), lambda i: (0, 0)),
            pl.BlockSpec((1, NY_PAD), lambda i: (0, 0)),
        ],
        out_specs=pl.BlockSpec((BLK_B, 1), lambda i: (i, 0)),
        out_shape=jax.ShapeDtypeStruct((BATCH, 1), jnp.float32),
        compiler_params=pltpu.CompilerParams(
            dimension_semantics=("arbitrary",),
        ),
    )(X, bt, y_pad)
    return out.reshape(BATCH)


# speedup vs baseline: 3.9588x; 1.0149x over previous
import functools

import jax
import jax.numpy as jnp
from jax.experimental import pallas as pl
from jax.experimental.pallas import tpu as pltpu

RADIUS = 2.0
Y_ACC = 0.001
X_ACC = 0.001
NUM_DIMS = 8
TEMP = 50.0
EPS = 0.0001
BATCH = 2048
NY = int(2 * RADIUS / Y_ACC) + 1
NX = int(2 * RADIUS / X_ACC) + 1
NY_PAD = 4096
L2E = 1.4426950408889634
BLK_B = 256


def _fsm_kernel(x_ref, b_ref, y_ref, out_ref):
    x = x_ref[...]
    xp = jnp.clip(x, -RADIUS + EPS, RADIUS - EPS)
    idx = jnp.round((xp + RADIUS) / (2.0 * RADIUS) * (NX - 1))
    xg = -RADIUS + idx * (2.0 * RADIUS / (NX - 1))

    y = y_ref[...]
    bt = b_ref[...]

    tl2e = TEMP * L2E
    ltm = tl2e * (1.0 - jnp.min(bt, axis=-1, keepdims=True))

    d = xg[:, :, None] - y[None, :, :]
    s = jnp.exp2(d * d * (-L2E)) - bt[None, :, :]
    e = jnp.exp2(s * tl2e - ltm[None, :, :])
    num = jnp.sum(e * s, axis=-1)
    den = jnp.sum(e, axis=-1)
    out_ref[...] = jnp.sum(num / den, axis=-1, keepdims=True)


@jax.jit
def kernel(X, theta):
    y_grid = jnp.linspace(-RADIUS, RADIUS, NY, dtype=jnp.float32)
    y_pad = jnp.zeros((1, NY_PAD), jnp.float32).at[0, :NY].set(y_grid)
    b = jnp.concatenate(
        [jnp.zeros((1, NUM_DIMS), jnp.float32), theta], axis=0
    )
    bt = jnp.full((NUM_DIMS, NY_PAD), 1e30, jnp.float32).at[:, :NY].set(b.T)

    grid = BATCH // BLK_B
    out = pl.pallas_call(
        _fsm_kernel,
        grid=(grid,),
        in_specs=[
            pl.BlockSpec((BLK_B, NUM_DIMS), lambda i: (i, 0)),
            pl.BlockSpec((NUM_DIMS, NY_PAD), lambda i: (0, 0)),
            pl.BlockSpec((1, NY_PAD), lambda i: (0, 0)),
        ],
        out_specs=pl.BlockSpec((BLK_B, 1), lambda i: (i, 0)),
        out_shape=jax.ShapeDtypeStruct((BATCH, 1), jnp.float32),
        compiler_params=pltpu.CompilerParams(
            dimension_semantics=("arbitrary",),
        ),
    )(X, bt, y_pad)
    return out.reshape(BATCH)


# speedup vs baseline: 4.1206x; 1.0409x over previous
import functools

import jax
import jax.numpy as jnp
from jax.experimental import pallas as pl
from jax.experimental.pallas import tpu as pltpu

RADIUS = 2.0
Y_ACC = 0.001
X_ACC = 0.001
NUM_DIMS = 8
TEMP = 50.0
EPS = 0.0001
BATCH = 2048
NY = int(2 * RADIUS / Y_ACC) + 1
NX = int(2 * RADIUS / X_ACC) + 1
NY_PAD = 4096
L2E = 1.4426950408889634
BLK_B = 256


def _fsm_kernel(x_ref, b_ref, out_ref):
    x = x_ref[...]
    xp = jnp.clip(x, -RADIUS + EPS, RADIUS - EPS)
    idx = jnp.round((xp + RADIUS) / (2.0 * RADIUS) * (NX - 1))
    xg = -RADIUS + idx * (2.0 * RADIUS / (NX - 1))

    y = (
        jax.lax.broadcasted_iota(jnp.int32, (1, NY_PAD), 1).astype(jnp.float32)
        * Y_ACC
        - RADIUS
    )
    bt = b_ref[...]

    tl2e = TEMP * L2E
    ltm = tl2e * (1.0 - jnp.min(bt, axis=-1, keepdims=True))

    d = xg[:, :, None] - y[None, :, :]
    s = jnp.exp2(jnp.square(d) * (-L2E)) - bt[None, :, :]
    e = jnp.exp2(s * tl2e - ltm[None, :, :])
    num = jnp.sum(e * s, axis=-1)
    den = jnp.sum(e, axis=-1)
    out_ref[...] = jnp.sum(num / den, axis=-1, keepdims=True)


@jax.jit
def kernel(X, theta):
    bt = jnp.concatenate(
        [
            jnp.zeros((NUM_DIMS, 1), jnp.float32),
            theta.T,
            jnp.full((NUM_DIMS, NY_PAD - NY), 1e30, jnp.float32),
        ],
        axis=1,
    )

    grid = BATCH // BLK_B
    out = pl.pallas_call(
        _fsm_kernel,
        grid=(grid,),
        in_specs=[
            pl.BlockSpec((BLK_B, NUM_DIMS), lambda i: (i, 0)),
            pl.BlockSpec((NUM_DIMS, NY_PAD), lambda i: (0, 0)),
        ],
        out_specs=pl.BlockSpec((BLK_B, 1), lambda i: (i, 0)),
        out_shape=jax.ShapeDtypeStruct((BATCH, 1), jnp.float32),
        compiler_params=pltpu.CompilerParams(
            dimension_semantics=("arbitrary",),
        ),
    )(X, bt)
    return out.reshape(BATCH)


# speedup vs baseline: 4.3463x; 1.0548x over previous
import functools

import jax
import jax.numpy as jnp
from jax.experimental import pallas as pl
from jax.experimental.pallas import tpu as pltpu

RADIUS = 2.0
Y_ACC = 0.001
X_ACC = 0.001
NUM_DIMS = 8
TEMP = 50.0
EPS = 0.0001
BATCH = 2048
NY = int(2 * RADIUS / Y_ACC) + 1
NX = int(2 * RADIUS / X_ACC) + 1
NY_PAD = 4096
L2E = 1.4426950408889634
BLK_B = 256


def _fsm_kernel(x_ref, b_ref, out_ref):
    x = x_ref[...]
    xp = jnp.clip(x, -RADIUS + EPS, RADIUS - EPS)
    idx = jnp.round((xp + RADIUS) / (2.0 * RADIUS) * (NX - 1))
    xg = -RADIUS + idx * (2.0 * RADIUS / (NX - 1))

    y = (
        jax.lax.broadcasted_iota(jnp.int32, (1, NY_PAD), 1).astype(jnp.float32)
        * Y_ACC
        - RADIUS
    )
    bt = b_ref[...]

    tl2e = TEMP * L2E
    ltm = tl2e * (1.0 - jnp.min(bt, axis=-1, keepdims=True))

    ch = 512
    nums, dens = [], []
    for k in range(NY_PAD // ch):
        yk = y[:, k * ch : (k + 1) * ch]
        btk = bt[:, k * ch : (k + 1) * ch]
        d = xg[:, :, None] - yk[None, :, :]
        s = jnp.exp2(jnp.square(d) * (-L2E)) - btk[None, :, :]
        e = jnp.exp2(s * tl2e - ltm[None, :, :])
        nums.append(jnp.sum(e * s, axis=-1))
        dens.append(jnp.sum(e, axis=-1))
    num = functools.reduce(jnp.add, nums)
    den = functools.reduce(jnp.add, dens)
    out_ref[...] = jnp.sum(num / den, axis=-1, keepdims=True)


@jax.jit
def kernel(X, theta):
    bt = jnp.concatenate(
        [
            jnp.zeros((NUM_DIMS, 1), jnp.float32),
            theta.T,
            jnp.full((NUM_DIMS, NY_PAD - NY), 1e30, jnp.float32),
        ],
        axis=1,
    )

    grid = BATCH // BLK_B
    out = pl.pallas_call(
        _fsm_kernel,
        grid=(grid,),
        in_specs=[
            pl.BlockSpec((BLK_B, NUM_DIMS), lambda i: (i, 0)),
            pl.BlockSpec((NUM_DIMS, NY_PAD), lambda i: (0, 0)),
        ],
        out_specs=pl.BlockSpec((BLK_B, 1), lambda i: (i, 0)),
        out_shape=jax.ShapeDtypeStruct((BATCH, 1), jnp.float32),
        compiler_params=pltpu.CompilerParams(
            dimension_semantics=("arbitrary",),
        ),
    )(X, bt)
    return out.reshape(BATCH)
